# Initial kernel scaffold; baseline (speedup 1.0000x reference)
import jax
import jax.numpy as jnp
from jax.experimental import pallas as pl


def kernel(pos, weight, batch):
    raise NotImplementedError("write your pallas kernel here")


# speedup vs baseline: 1.5487x; 1.5487x over previous
import jax
import jax.numpy as jnp
from jax.experimental import pallas as pl

N = 50000
V = 128
G = 256
EPS = 1e-05
BLK = 1000
NB = N // BLK


def _seg_kernel(pos_ref, batch_ref, seg_ref, cnt_ref):
    i = pl.program_id(0)
    x = pos_ref[...]
    nrm = jnp.sqrt(x[:, :V] * x[:, :V] + x[:, V:2 * V] * x[:, V:2 * V]
                   + x[:, 2 * V:] * x[:, 2 * V:])
    b = batch_ref[0, 0, :]
    oh = (jax.lax.broadcasted_iota(jnp.int32, (G, BLK), 0)
          == b[None, :]).astype(jnp.float32)
    part = jnp.dot(oh, nrm, preferred_element_type=jnp.float32)
    pcnt = jnp.sum(oh, axis=1)[None, :]

    @pl.when(i == 0)
    def _():
        seg_ref[...] = jnp.zeros_like(seg_ref)
        cnt_ref[...] = jnp.zeros_like(cnt_ref)

    seg_ref[...] += part
    cnt_ref[...] += pcnt


def _norm_kernel(pos_ref, batch_ref, seg_ref, cnt_ref, w_ref, out_ref):
    x = pos_ref[...]
    b = batch_ref[0, 0, :]
    cnt = jnp.maximum(cnt_ref[0, :], 1.0)
    mean = seg_ref[...] / cnt[:, None]
    oh = (b[:, None] == jax.lax.broadcasted_iota(jnp.int32, (BLK, G), 1)
          ).astype(jnp.float32)
    gm = jnp.dot(oh, mean, preferred_element_type=jnp.float32)
    denom = gm + EPS
    w = w_ref[0, :]
    out_ref[:, :V] = x[:, :V] * w[None, :] / denom
    out_ref[:, V:2 * V] = x[:, V:2 * V] * w[None, :] / denom
    out_ref[:, 2 * V:] = x[:, 2 * V:] * w[None, :] / denom


def kernel(pos, weight, batch):
    posf = pos.reshape(N, 3 * V)
    b3 = batch.astype(jnp.int32).reshape(NB, 1, BLK)
    wf = weight.reshape(1, V)

    seg, cnt = pl.pallas_call(
        _seg_kernel,
        grid=(NB,),
        in_specs=[
            pl.BlockSpec((BLK, 3 * V), lambda i: (i, 0)),
            pl.BlockSpec((1, 1, BLK), lambda i: (i, 0, 0)),
        ],
        out_specs=[
            pl.BlockSpec((G, V), lambda i: (0, 0)),
            pl.BlockSpec((1, G), lambda i: (0, 0)),
        ],
        out_shape=[
            jax.ShapeDtypeStruct((G, V), jnp.float32),
            jax.ShapeDtypeStruct((1, G), jnp.float32),
        ],
    )(posf, b3)

    out = pl.pallas_call(
        _norm_kernel,
        grid=(NB,),
        in_specs=[
            pl.BlockSpec((BLK, 3 * V), lambda i: (i, 0)),
            pl.BlockSpec((1, 1, BLK), lambda i: (i, 0, 0)),
            pl.BlockSpec((G, V), lambda i: (0, 0)),
            pl.BlockSpec((1, G), lambda i: (0, 0)),
            pl.BlockSpec((1, V), lambda i: (0, 0)),
        ],
        out_specs=pl.BlockSpec((BLK, 3 * V), lambda i: (i, 0)),
        out_shape=jax.ShapeDtypeStruct((N, 3 * V), jnp.float32),
    )(posf, b3, seg, cnt, wf)

    return out.reshape(N, 3, V)
